# baseline (speedup 1.0000x reference)
import functools

import jax
import jax.numpy as jnp
from jax import lax
from jax.experimental import pallas as pl
from jax.experimental.pallas import tpu as pltpu
from jax.experimental.pallas import tpu_sc as plsc

N, E, D, DE, H, K = 10000, 160000, 128, 16, 256, 16
TE = 2000
TN = 2000
NC, NS = 2, 16
NW = NC * NS
EPW = E // NW
CH = 1000
NCHUNK = EPW // CH
NPT = N // NS


def _elu(z):
    return jnp.where(z > 0, z, jnp.exp(z) - 1.0)


def _xp_body(x_ref, win_ref, bin_ref, out_ref):
    out_ref[...] = _elu(
        jnp.dot(x_ref[...], win_ref[...], preferred_element_type=jnp.float32)
        + bin_ref[...])


def _edge_msg_body(ea_ref, gs_ref, gd_ref, w1a_ref, w1b32_ref, w1b_ref,
                   b1_ref, w2_ref, b2_ref, r32_ref, s_ref, out_ref):
    gs = gs_ref[...]
    z = (jnp.dot(ea_ref[...], w1a_ref[...], preferred_element_type=jnp.float32)
         + jnp.dot(gs, w1b32_ref[...], preferred_element_type=jnp.float32)
         - jnp.dot(gd_ref[...], w1b_ref[...], preferred_element_type=jnp.float32)
         + b1_ref[...])
    h1 = _elu(z)
    kern = jnp.dot(h1, w2_ref[...], preferred_element_type=jnp.float32) + b2_ref[...]
    xrep = jnp.dot(gs, r32_ref[...], preferred_element_type=jnp.float32)
    out_ref[...] = jnp.dot(kern * xrep, s_ref[...],
                           preferred_element_type=jnp.float32)


def _node_body(xp_ref, agg_ref, wroot_ref, broot_ref, wn1_ref, bn1_ref,
               wn2_ref, bn2_ref, hn_ref, np_ref):
    xp = xp_ref[...]
    agg = agg_ref[0] + agg_ref[1]
    hn = _elu(jnp.dot(xp, wroot_ref[...], preferred_element_type=jnp.float32)
              + broot_ref[...] + agg)
    t = _elu(jnp.dot(hn, wn1_ref[...], preferred_element_type=jnp.float32)
             + bn1_ref[...])
    hn_ref[...] = hn
    np_ref[...] = jnp.dot(t, wn2_ref[...], preferred_element_type=jnp.float32) \
        + bn2_ref[...]


def _edge_pred_body(hs_ref, hd_ref, we1a_ref, we1b_ref, be1_ref,
                    we2_ref, be2_ref, out_ref):
    t = _elu(jnp.dot(hs_ref[...], we1a_ref[...], preferred_element_type=jnp.float32)
             + jnp.dot(hd_ref[...], we1b_ref[...], preferred_element_type=jnp.float32)
             + be1_ref[...])
    out_ref[...] = jnp.dot(t, we2_ref[...], preferred_element_type=jnp.float32) \
        + be2_ref[...]


def _full(shape):
    return pl.BlockSpec(shape, lambda i: (0,) * len(shape))


_SC_MESH = plsc.VectorSubcoreMesh(core_axis_name="c", subcore_axis_name="s")
_SC_PARAMS = pltpu.CompilerParams(use_tc_tiling_on_sc=False)


def _make_gather2(wa, wb):
    def body(taba_hbm, tabb_hbm, idxa_hbm, idxb_hbm, outa_hbm, outb_hbm,
             ia_v, ib_v, ra_v, rb_v,
             sga0, sga1, sgb0, sgb1, swa0, swa1, swb0, swb1):
        wid = lax.axis_index("s") * NC + lax.axis_index("c")
        base0 = wid * EPW
        pltpu.sync_copy(idxa_hbm.at[pl.ds(base0, EPW)], ia_v)
        pltpu.sync_copy(idxb_hbm.at[pl.ds(base0, EPW)], ib_v)
        sga = (sga0, sga1)
        sgb = (sgb0, sgb1)
        swa = (swa0, swa1)
        swb = (swb0, swb1)
        ga = [None] * NCHUNK
        gb = [None] * NCHUNK
        wa_ = [None] * NCHUNK
        wb_ = [None] * NCHUNK
        for j in range(NCHUNK):
            b = j % 2
            if j >= 2:
                wa_[j - 2].wait()
                wb_[j - 2].wait()
            ga[j] = pltpu.async_copy(
                taba_hbm.at[ia_v.at[pl.ds(j * CH, CH)]], ra_v.at[b], sga[b])
            gb[j] = pltpu.async_copy(
                tabb_hbm.at[ib_v.at[pl.ds(j * CH, CH)]], rb_v.at[b], sgb[b])
            if j >= 1:
                p = (j - 1) % 2
                ga[j - 1].wait()
                gb[j - 1].wait()
                wa_[j - 1] = pltpu.async_copy(
                    ra_v.at[p], outa_hbm.at[pl.ds(base0 + (j - 1) * CH, CH)],
                    swa[p])
                wb_[j - 1] = pltpu.async_copy(
                    rb_v.at[p], outb_hbm.at[pl.ds(base0 + (j - 1) * CH, CH)],
                    swb[p])
        jl = NCHUNK - 1
        b = jl % 2
        ga[jl].wait()
        gb[jl].wait()
        wa_[jl] = pltpu.async_copy(
            ra_v.at[b], outa_hbm.at[pl.ds(base0 + jl * CH, CH)], swa[b])
        wb_[jl] = pltpu.async_copy(
            rb_v.at[b], outb_hbm.at[pl.ds(base0 + jl * CH, CH)], swb[b])
        wa_[jl - 1].wait()
        wb_[jl - 1].wait()
        wa_[jl].wait()
        wb_[jl].wait()

    return pl.kernel(
        body,
        out_type=[jax.ShapeDtypeStruct((E, wa), jnp.float32),
                  jax.ShapeDtypeStruct((E, wb), jnp.float32)],
        mesh=_SC_MESH,
        scratch_types=[pltpu.VMEM((EPW,), jnp.int32),
                       pltpu.VMEM((EPW,), jnp.int32),
                       pltpu.VMEM((2, CH, wa), jnp.float32),
                       pltpu.VMEM((2, CH, wb), jnp.float32)]
        + [pltpu.SemaphoreType.DMA] * 8,
        compiler_params=_SC_PARAMS,
    )


def _seg_sum_body(msg_hbm, idx_hbm, zeros_hbm, out_hbm, idx_v, msg_v, acc_sh,
                  sl0, sl1, ss0, ss1):
    cid = lax.axis_index("c")
    sid = lax.axis_index("s")
    wid = sid * NC + cid
    base0 = wid * EPW
    pltpu.sync_copy(zeros_hbm.at[pl.ds(sid * NPT, NPT)],
                    acc_sh.at[pl.ds(sid * NPT, NPT)])
    pltpu.sync_copy(idx_hbm.at[pl.ds(base0, EPW)], idx_v)
    plsc.subcore_barrier()
    sl = (sl0, sl1)
    ss = (ss0, ss1)
    ld = [None] * NCHUNK
    sc = [None] * NCHUNK
    for j in range(NCHUNK):
        b = j % 2
        if j >= 2:
            sc[j - 2].wait()
        ld[j] = pltpu.async_copy(
            msg_hbm.at[pl.ds(base0 + j * CH, CH)], msg_v.at[b], sl[b])
        if j >= 1:
            p = (j - 1) % 2
            ld[j - 1].wait()
            sc[j - 1] = pltpu.async_copy(
                msg_v.at[p], acc_sh.at[idx_v.at[pl.ds((j - 1) * CH, CH)]],
                ss[p], add=True)
    jl = NCHUNK - 1
    ld[jl].wait()
    sc[jl] = pltpu.async_copy(
        msg_v.at[jl % 2], acc_sh.at[idx_v.at[pl.ds(jl * CH, CH)]],
        ss[jl % 2], add=True)
    sc[jl - 1].wait()
    sc[jl].wait()
    plsc.subcore_barrier()
    pltpu.sync_copy(acc_sh.at[pl.ds(sid * NPT, NPT)],
                    out_hbm.at[cid, pl.ds(sid * NPT, NPT)])


_seg_sum = pl.kernel(
    _seg_sum_body,
    out_type=jax.ShapeDtypeStruct((NC, N, K), jnp.float32),
    mesh=_SC_MESH,
    scratch_types=[pltpu.VMEM((EPW,), jnp.int32),
                   pltpu.VMEM((2, CH, K), jnp.float32),
                   pltpu.VMEM_SHARED((N, K), jnp.float32)]
    + [pltpu.SemaphoreType.DMA] * 4,
    compiler_params=_SC_PARAMS,
)


def kernel(x, edge_index, edge_attr, batch, pos, W1, b1, W2, b2, Win, bin_,
           Wroot, broot, Wn1, bn1, Wn2, bn2, We1, be1, We2, be2):
    f32 = jnp.float32
    src = edge_index[0]
    dst = edge_index[1]

    xp = pl.pallas_call(
        _xp_body,
        grid=(N // TN,),
        in_specs=[pl.BlockSpec((TN, D), lambda i: (i, 0)),
                  _full((D, K)), _full((1, K))],
        out_specs=pl.BlockSpec((TN, K), lambda i: (i, 0)),
        out_shape=jax.ShapeDtypeStruct((N, K), f32),
    )(x, Win, bin_.reshape(1, K))

    pos_pad = jnp.pad(pos, ((0, 0), (0, 13)))
    tab = jnp.concatenate([xp, pos_pad], axis=1)
    gs, gd = _make_gather2(2 * K, K)(tab, pos_pad, src, dst)

    r_sel = (jnp.arange(H)[None, :] // K == jnp.arange(K)[:, None]).astype(f32)
    r32_sel = jnp.pad(r_sel, ((0, K), (0, 0)))
    s_sel = (jnp.arange(H)[:, None] % K == jnp.arange(K)[None, :]).astype(f32)

    W1a = W1[:DE]
    W1b = jnp.pad(W1[DE:], ((0, 13), (0, 0)))
    W1b32 = jnp.pad(W1[DE:], ((DE, 10), (0, 0)))

    msg = pl.pallas_call(
        _edge_msg_body,
        grid=(E // TE,),
        in_specs=[pl.BlockSpec((TE, DE), lambda i: (i, 0)),
                  pl.BlockSpec((TE, 2 * K), lambda i: (i, 0)),
                  pl.BlockSpec((TE, K), lambda i: (i, 0)),
                  _full((DE, H)), _full((2 * K, H)), _full((K, H)),
                  _full((1, H)),
                  _full((H, K * K)), _full((1, K * K)),
                  _full((2 * K, H)), _full((H, K))],
        out_specs=pl.BlockSpec((TE, K), lambda i: (i, 0)),
        out_shape=jax.ShapeDtypeStruct((E, K), f32),
    )(edge_attr, gs, gd, W1a, W1b32, W1b, b1.reshape(1, H), W2,
      b2.reshape(1, K * K), r32_sel, s_sel)

    agg2 = _seg_sum(msg, dst, jnp.zeros((N, K), f32))

    hn, node_pred = pl.pallas_call(
        _node_body,
        grid=(N // TN,),
        in_specs=[pl.BlockSpec((TN, K), lambda i: (i, 0)),
                  pl.BlockSpec((NC, TN, K), lambda i: (0, i, 0)),
                  _full((K, K)), _full((1, K)),
                  _full((K, 64)), _full((1, 64)),
                  _full((64, 2)), _full((1, 2))],
        out_specs=[pl.BlockSpec((TN, K), lambda i: (i, 0)),
                   pl.BlockSpec((TN, 2), lambda i: (i, 0))],
        out_shape=[jax.ShapeDtypeStruct((N, K), f32),
                   jax.ShapeDtypeStruct((N, 2), f32)],
    )(xp, agg2, Wroot, broot.reshape(1, K), Wn1, bn1.reshape(1, 64),
      Wn2, bn2.reshape(1, 2))

    hs, hd = _make_gather2(K, K)(hn, hn, src, dst)
    edge_pred = pl.pallas_call(
        _edge_pred_body,
        grid=(E // TE,),
        in_specs=[pl.BlockSpec((TE, K), lambda i: (i, 0)),
                  pl.BlockSpec((TE, K), lambda i: (i, 0)),
                  _full((K, 64)), _full((K, 64)), _full((1, 64)),
                  _full((64, 2)), _full((1, 2))],
        out_specs=pl.BlockSpec((TE, 2), lambda i: (i, 0)),
        out_shape=jax.ShapeDtypeStruct((E, 2), f32),
    )(hs, hd, We1[:K], We1[K:], be1.reshape(1, 64), We2, be2.reshape(1, 2))

    return node_pred, edge_pred


# speedup vs baseline: 1.0067x; 1.0067x over previous
import functools

import jax
import jax.numpy as jnp
from jax import lax
from jax.experimental import pallas as pl
from jax.experimental.pallas import tpu as pltpu
from jax.experimental.pallas import tpu_sc as plsc

N, E, D, DE, H, K = 10000, 160000, 128, 16, 256, 16
EH1, EH2 = 81920, 78080
CH1, CH2 = 512, 488
TE1, TE2 = 2048, 1952
NCHUNK = 5
TN = 2000
NC, NS = 2, 16
NW = NC * NS
NPT = N // NS


def _elu(z):
    return jnp.where(z > 0, z, jnp.exp(z) - 1.0)


def _xp_body(x_ref, win_ref, bin_ref, out_ref):
    out_ref[...] = _elu(
        jnp.dot(x_ref[...], win_ref[...], preferred_element_type=jnp.float32)
        + bin_ref[...])


def _edge_msg_body(ea_ref, gs_ref, gd_ref, w1a_ref, w1b32_ref, w1b_ref,
                   b1_ref, w2_ref, b2_ref, r32_ref, s_ref, out_ref):
    gs = gs_ref[...]
    z = (jnp.dot(ea_ref[...], w1a_ref[...], preferred_element_type=jnp.float32)
         + jnp.dot(gs, w1b32_ref[...], preferred_element_type=jnp.float32)
         - jnp.dot(gd_ref[...], w1b_ref[...], preferred_element_type=jnp.float32)
         + b1_ref[...])
    h1 = _elu(z)
    kern = jnp.dot(h1, w2_ref[...], preferred_element_type=jnp.float32) + b2_ref[...]
    xrep = jnp.dot(gs, r32_ref[...], preferred_element_type=jnp.float32)
    out_ref[...] = jnp.dot(kern * xrep, s_ref[...],
                           preferred_element_type=jnp.float32)


def _node_body(xp_ref, a1_ref, a2_ref, wroot_ref, broot_ref, wn1_ref, bn1_ref,
               wn2_ref, bn2_ref, hn_ref, np_ref):
    xp = xp_ref[...]
    agg = a1_ref[0] + a1_ref[1] + a2_ref[0] + a2_ref[1]
    hn = _elu(jnp.dot(xp, wroot_ref[...], preferred_element_type=jnp.float32)
              + broot_ref[...] + agg)
    t = _elu(jnp.dot(hn, wn1_ref[...], preferred_element_type=jnp.float32)
             + bn1_ref[...])
    hn_ref[...] = hn
    np_ref[...] = jnp.dot(t, wn2_ref[...], preferred_element_type=jnp.float32) \
        + bn2_ref[...]


def _edge_pred_body(hs_ref, hd_ref, we1a_ref, we1b_ref, be1_ref,
                    we2_ref, be2_ref, out_ref):
    t = _elu(jnp.dot(hs_ref[...], we1a_ref[...], preferred_element_type=jnp.float32)
             + jnp.dot(hd_ref[...], we1b_ref[...], preferred_element_type=jnp.float32)
             + be1_ref[...])
    out_ref[...] = jnp.dot(t, we2_ref[...], preferred_element_type=jnp.float32) \
        + be2_ref[...]


def _full(shape):
    return pl.BlockSpec(shape, lambda i: (0,) * len(shape))


_SC_MESH = plsc.VectorSubcoreMesh(core_axis_name="c", subcore_axis_name="s")
_SC_PARAMS = pltpu.CompilerParams(use_tc_tiling_on_sc=False)


def _make_gather2(wa, wb, ne, ch):
    epw = ne // NW

    def body(taba_hbm, tabb_hbm, idxa_hbm, idxb_hbm, outa_hbm, outb_hbm,
             ia_v, ib_v, ra_v, rb_v,
             sga0, sga1, sgb0, sgb1, swa0, swa1, swb0, swb1):
        wid = lax.axis_index("s") * NC + lax.axis_index("c")
        sga = (sga0, sga1)
        sgb = (sgb0, sgb1)
        swa = (swa0, swa1)
        swb = (swb0, swb1)
        ga = [None] * NCHUNK
        gb = [None] * NCHUNK
        wa_ = [None] * NCHUNK
        wb_ = [None] * NCHUNK

        def base(j):
            return wid * epw + j * ch

        for j in range(NCHUNK):
            b = j % 2
            pltpu.sync_copy(idxa_hbm.at[pl.ds(base(j), ch)], ia_v.at[b])
            pltpu.sync_copy(idxb_hbm.at[pl.ds(base(j), ch)], ib_v.at[b])
            if j >= 2:
                wa_[j - 2].wait()
                wb_[j - 2].wait()
            ga[j] = pltpu.async_copy(taba_hbm.at[ia_v.at[b]], ra_v.at[b],
                                     sga[b])
            gb[j] = pltpu.async_copy(tabb_hbm.at[ib_v.at[b]], rb_v.at[b],
                                     sgb[b])
            if j >= 1:
                p = (j - 1) % 2
                ga[j - 1].wait()
                gb[j - 1].wait()
                wa_[j - 1] = pltpu.async_copy(
                    ra_v.at[p], outa_hbm.at[pl.ds(base(j - 1), ch)], swa[p])
                wb_[j - 1] = pltpu.async_copy(
                    rb_v.at[p], outb_hbm.at[pl.ds(base(j - 1), ch)], swb[p])
        jl = NCHUNK - 1
        b = jl % 2
        ga[jl].wait()
        gb[jl].wait()
        wa_[jl] = pltpu.async_copy(
            ra_v.at[b], outa_hbm.at[pl.ds(base(jl), ch)], swa[b])
        wb_[jl] = pltpu.async_copy(
            rb_v.at[b], outb_hbm.at[pl.ds(base(jl), ch)], swb[b])
        wa_[jl - 1].wait()
        wb_[jl - 1].wait()
        wa_[jl].wait()
        wb_[jl].wait()

    return pl.kernel(
        body,
        out_type=[jax.ShapeDtypeStruct((ne, wa), jnp.float32),
                  jax.ShapeDtypeStruct((ne, wb), jnp.float32)],
        mesh=_SC_MESH,
        scratch_types=[pltpu.VMEM((2, ch), jnp.int32),
                       pltpu.VMEM((2, ch), jnp.int32),
                       pltpu.VMEM((2, ch, wa), jnp.float32),
                       pltpu.VMEM((2, ch, wb), jnp.float32)]
        + [pltpu.SemaphoreType.DMA] * 8,
        compiler_params=_SC_PARAMS,
    )


def _make_seg_sum(ne, ch):
    epw = ne // NW

    def body(msg_hbm, idx_hbm, zeros_hbm, out_hbm, idx_v, msg_v, acc_sh,
             sl0, sl1, ss0, ss1):
        cid = lax.axis_index("c")
        sid = lax.axis_index("s")
        wid = sid * NC + cid
        pltpu.sync_copy(zeros_hbm.at[pl.ds(sid * NPT, NPT)],
                        acc_sh.at[pl.ds(sid * NPT, NPT)])
        plsc.subcore_barrier()
        sl = (sl0, sl1)
        ss = (ss0, ss1)
        ld = [None] * NCHUNK
        sc = [None] * NCHUNK

        def base(j):
            return wid * epw + j * ch

        for j in range(NCHUNK):
            b = j % 2
            pltpu.sync_copy(idx_hbm.at[pl.ds(base(j), ch)], idx_v.at[b])
            if j >= 2:
                sc[j - 2].wait()
            ld[j] = pltpu.async_copy(
                msg_hbm.at[pl.ds(base(j), ch)], msg_v.at[b], sl[b])
            if j >= 1:
                p = (j - 1) % 2
                ld[j - 1].wait()
                sc[j - 1] = pltpu.async_copy(
                    msg_v.at[p], acc_sh.at[idx_v.at[p]], ss[p], add=True)
        jl = NCHUNK - 1
        ld[jl].wait()
        sc[jl] = pltpu.async_copy(
            msg_v.at[jl % 2], acc_sh.at[idx_v.at[jl % 2]], ss[jl % 2],
            add=True)
        sc[jl - 1].wait()
        sc[jl].wait()
        plsc.subcore_barrier()
        pltpu.sync_copy(acc_sh.at[pl.ds(sid * NPT, NPT)],
                        out_hbm.at[cid, pl.ds(sid * NPT, NPT)])

    return pl.kernel(
        body,
        out_type=jax.ShapeDtypeStruct((NC, N, K), jnp.float32),
        mesh=_SC_MESH,
        scratch_types=[pltpu.VMEM((2, ch), jnp.int32),
                       pltpu.VMEM((2, ch, K), jnp.float32),
                       pltpu.VMEM_SHARED((N, K), jnp.float32)]
        + [pltpu.SemaphoreType.DMA] * 4,
        compiler_params=_SC_PARAMS,
    )


def kernel(x, edge_index, edge_attr, batch, pos, W1, b1, W2, b2, Win, bin_,
           Wroot, broot, Wn1, bn1, Wn2, bn2, We1, be1, We2, be2):
    f32 = jnp.float32
    src = edge_index[0]
    dst = edge_index[1]
    src1, src2 = src[:EH1], src[EH1:]
    dst1, dst2 = dst[:EH1], dst[EH1:]

    xp = pl.pallas_call(
        _xp_body,
        grid=(N // TN,),
        in_specs=[pl.BlockSpec((TN, D), lambda i: (i, 0)),
                  _full((D, K)), _full((1, K))],
        out_specs=pl.BlockSpec((TN, K), lambda i: (i, 0)),
        out_shape=jax.ShapeDtypeStruct((N, K), f32),
    )(x, Win, bin_.reshape(1, K))

    pos_pad = jnp.pad(pos, ((0, 0), (0, 13)))
    tab = jnp.concatenate([xp, pos_pad], axis=1)
    gs1, gd1 = _make_gather2(2 * K, K, EH1, CH1)(tab, pos_pad, src1, dst1)
    gs2, gd2 = _make_gather2(2 * K, K, EH2, CH2)(tab, pos_pad, src2, dst2)

    r_sel = (jnp.arange(H)[None, :] // K == jnp.arange(K)[:, None]).astype(f32)
    r32_sel = jnp.pad(r_sel, ((0, K), (0, 0)))
    s_sel = (jnp.arange(H)[:, None] % K == jnp.arange(K)[None, :]).astype(f32)

    W1a = W1[:DE]
    W1b = jnp.pad(W1[DE:], ((0, 13), (0, 0)))
    W1b32 = jnp.pad(W1[DE:], ((DE, 10), (0, 0)))

    def edge_msg(ea, gs, gd, ne, te):
        return pl.pallas_call(
            _edge_msg_body,
            grid=(ne // te,),
            in_specs=[pl.BlockSpec((te, DE), lambda i: (i, 0)),
                      pl.BlockSpec((te, 2 * K), lambda i: (i, 0)),
                      pl.BlockSpec((te, K), lambda i: (i, 0)),
                      _full((DE, H)), _full((2 * K, H)), _full((K, H)),
                      _full((1, H)),
                      _full((H, K * K)), _full((1, K * K)),
                      _full((2 * K, H)), _full((H, K))],
            out_specs=pl.BlockSpec((te, K), lambda i: (i, 0)),
            out_shape=jax.ShapeDtypeStruct((ne, K), f32),
        )(ea, gs, gd, W1a, W1b32, W1b, b1.reshape(1, H), W2,
          b2.reshape(1, K * K), r32_sel, s_sel)

    msg1 = edge_msg(edge_attr[:EH1], gs1, gd1, EH1, TE1)
    msg2 = edge_msg(edge_attr[EH1:], gs2, gd2, EH2, TE2)

    zeros = jnp.zeros((N, K), f32)
    agg1 = _make_seg_sum(EH1, CH1)(msg1, dst1, zeros)
    agg2 = _make_seg_sum(EH2, CH2)(msg2, dst2, zeros)

    hn, node_pred = pl.pallas_call(
        _node_body,
        grid=(N // TN,),
        in_specs=[pl.BlockSpec((TN, K), lambda i: (i, 0)),
                  pl.BlockSpec((NC, TN, K), lambda i: (0, i, 0)),
                  pl.BlockSpec((NC, TN, K), lambda i: (0, i, 0)),
                  _full((K, K)), _full((1, K)),
                  _full((K, 64)), _full((1, 64)),
                  _full((64, 2)), _full((1, 2))],
        out_specs=[pl.BlockSpec((TN, K), lambda i: (i, 0)),
                   pl.BlockSpec((TN, 2), lambda i: (i, 0))],
        out_shape=[jax.ShapeDtypeStruct((N, K), f32),
                   jax.ShapeDtypeStruct((N, 2), f32)],
    )(xp, agg1, agg2, Wroot, broot.reshape(1, K), Wn1, bn1.reshape(1, 64),
      Wn2, bn2.reshape(1, 2))

    def edge_pred_half(hs, hd, ne, te):
        return pl.pallas_call(
            _edge_pred_body,
            grid=(ne // te,),
            in_specs=[pl.BlockSpec((te, K), lambda i: (i, 0)),
                      pl.BlockSpec((te, K), lambda i: (i, 0)),
                      _full((K, 64)), _full((K, 64)), _full((1, 64)),
                      _full((64, 2)), _full((1, 2))],
            out_specs=pl.BlockSpec((te, 2), lambda i: (i, 0)),
            out_shape=jax.ShapeDtypeStruct((ne, 2), f32),
        )(hs, hd, We1[:K], We1[K:], be1.reshape(1, 64), We2,
          be2.reshape(1, 2))

    hs1, hd1 = _make_gather2(K, K, EH1, CH1)(hn, hn, src1, dst1)
    hs2, hd2 = _make_gather2(K, K, EH2, CH2)(hn, hn, src2, dst2)
    ep1 = edge_pred_half(hs1, hd1, EH1, TE1)
    ep2 = edge_pred_half(hs2, hd2, EH2, TE2)
    edge_pred = jnp.concatenate([ep1, ep2], axis=0)

    return node_pred, edge_pred


# speedup vs baseline: 1.0099x; 1.0032x over previous
import functools

import jax
import jax.numpy as jnp
from jax import lax
from jax.experimental import pallas as pl
from jax.experimental.pallas import tpu as pltpu
from jax.experimental.pallas import tpu_sc as plsc

N, E, D, DE, H, K = 10000, 160000, 128, 16, 256, 16
EH1, EH2 = 81920, 78080
CH1, CH2 = 512, 488
TE1, TE2 = 2048, 1952
NCHUNK = 5
TN = 2000
NC, NS = 2, 16
NW = NC * NS
NPT = N // NS


def _elu(z):
    return jnp.where(z > 0, z, jnp.exp(z) - 1.0)


def _xp_body(x_ref, win_ref, bin_ref, out_ref):
    out_ref[...] = _elu(
        jnp.dot(x_ref[...], win_ref[...], preferred_element_type=jnp.float32)
        + bin_ref[...])


def _edge_msg_body(ea_ref, gs_ref, gd_ref, w1a_ref, w1b32_ref, w1b_ref,
                   b1_ref, w2_ref, b2_ref, r32_ref, s_ref, out_ref):
    gs = gs_ref[...]
    z = (jnp.dot(ea_ref[...], w1a_ref[...], preferred_element_type=jnp.float32)
         + jnp.dot(gs, w1b32_ref[...], preferred_element_type=jnp.float32)
         - jnp.dot(gd_ref[...], w1b_ref[...], preferred_element_type=jnp.float32)
         + b1_ref[...])
    h1 = _elu(z)
    kern = jnp.dot(h1, w2_ref[...], preferred_element_type=jnp.float32) + b2_ref[...]
    xrep = jnp.dot(gs, r32_ref[...], preferred_element_type=jnp.float32)
    out_ref[...] = jnp.dot(kern * xrep, s_ref[...],
                           preferred_element_type=jnp.float32)


def _node_body(xp_ref, a1_ref, a2_ref, wroot_ref, broot_ref, wn1_ref, bn1_ref,
               wn2_ref, bn2_ref, hn_ref, np_ref):
    xp = xp_ref[...]
    agg = a1_ref[0] + a1_ref[1] + a2_ref[0] + a2_ref[1]
    hn = _elu(jnp.dot(xp, wroot_ref[...], preferred_element_type=jnp.float32)
              + broot_ref[...] + agg)
    t = _elu(jnp.dot(hn, wn1_ref[...], preferred_element_type=jnp.float32)
             + bn1_ref[...])
    hn_ref[...] = hn
    np_ref[...] = jnp.dot(t, wn2_ref[...], preferred_element_type=jnp.float32) \
        + bn2_ref[...]


def _edge_pred_body(hs_ref, hd_ref, we1a_ref, we1b_ref, be1_ref,
                    we2_ref, be2_ref, out_ref):
    t = _elu(jnp.dot(hs_ref[...], we1a_ref[...], preferred_element_type=jnp.float32)
             + jnp.dot(hd_ref[...], we1b_ref[...], preferred_element_type=jnp.float32)
             + be1_ref[...])
    out_ref[...] = jnp.dot(t, we2_ref[...], preferred_element_type=jnp.float32) \
        + be2_ref[...]


def _full(shape):
    return pl.BlockSpec(shape, lambda i: (0,) * len(shape))


_SC_MESH = plsc.VectorSubcoreMesh(core_axis_name="c", subcore_axis_name="s")
_SC_PARAMS = pltpu.CompilerParams(use_tc_tiling_on_sc=False)


def _make_gather2(wa, wb, ne, ch):
    epw = ne // NW

    def body(taba_hbm, tabb_hbm, idxa_hbm, idxb_hbm, outa_hbm, outb_hbm,
             ia_v, ib_v, ra_v, rb_v,
             sga0, sga1, sga2, sga3, sgb0, sgb1, sgb2, sgb3,
             swa0, swa1, swa2, swa3, swb0, swb1, swb2, swb3):
        wid = lax.axis_index("s") * NC + lax.axis_index("c")
        sga = (sga0, sga1, sga2, sga3)
        sgb = (sgb0, sgb1, sgb2, sgb3)
        swa = (swa0, swa1, swa2, swa3)
        swb = (swb0, swb1, swb2, swb3)
        ga = [None] * NCHUNK
        gb = [None] * NCHUNK
        wa_ = [None] * NCHUNK
        wb_ = [None] * NCHUNK
        base0 = wid * epw
        pltpu.sync_copy(idxa_hbm.at[pl.ds(base0, epw)], ia_v)
        pltpu.sync_copy(idxb_hbm.at[pl.ds(base0, epw)], ib_v)

        def start_write(j):
            b = j % 4
            ga[j].wait()
            gb[j].wait()
            wa_[j] = pltpu.async_copy(
                ra_v.at[b], outa_hbm.at[pl.ds(base0 + j * ch, ch)], swa[b])
            wb_[j] = pltpu.async_copy(
                rb_v.at[b], outb_hbm.at[pl.ds(base0 + j * ch, ch)], swb[b])

        for j in range(NCHUNK):
            b = j % 4
            if j >= 4:
                wa_[j - 4].wait()
                wb_[j - 4].wait()
            ga[j] = pltpu.async_copy(
                taba_hbm.at[ia_v.at[pl.ds(j * ch, ch)]], ra_v.at[b], sga[b])
            gb[j] = pltpu.async_copy(
                tabb_hbm.at[ib_v.at[pl.ds(j * ch, ch)]], rb_v.at[b], sgb[b])
            if j >= 2:
                start_write(j - 2)
        for j in range(max(NCHUNK - 2, 0), NCHUNK):
            start_write(j)
        for j in range(max(NCHUNK - 4, 0), NCHUNK):
            wa_[j].wait()
            wb_[j].wait()

    return pl.kernel(
        body,
        out_type=[jax.ShapeDtypeStruct((ne, wa), jnp.float32),
                  jax.ShapeDtypeStruct((ne, wb), jnp.float32)],
        mesh=_SC_MESH,
        scratch_types=[pltpu.VMEM((epw,), jnp.int32),
                       pltpu.VMEM((epw,), jnp.int32),
                       pltpu.VMEM((4, ch, wa), jnp.float32),
                       pltpu.VMEM((4, ch, wb), jnp.float32)]
        + [pltpu.SemaphoreType.DMA] * 16,
        compiler_params=_SC_PARAMS,
    )


def _make_seg_sum(ne, ch):
    epw = ne // NW

    def body(msg_hbm, idx_hbm, zeros_hbm, out_hbm, idx_v, msg_v, acc_sh,
             sl0, sl1, sl2, sl3, ss0, ss1, ss2, ss3):
        cid = lax.axis_index("c")
        sid = lax.axis_index("s")
        wid = sid * NC + cid
        base0 = wid * epw
        pltpu.sync_copy(zeros_hbm.at[pl.ds(sid * NPT, NPT)],
                        acc_sh.at[pl.ds(sid * NPT, NPT)])
        pltpu.sync_copy(idx_hbm.at[pl.ds(base0, epw)], idx_v)
        plsc.subcore_barrier()
        sl = (sl0, sl1, sl2, sl3)
        ss = (ss0, ss1, ss2, ss3)
        ld = [None] * NCHUNK
        sc = [None] * NCHUNK

        def start_scatter(j):
            b = j % 4
            ld[j].wait()
            sc[j] = pltpu.async_copy(
                msg_v.at[b], acc_sh.at[idx_v.at[pl.ds(j * ch, ch)]], ss[b],
                add=True)

        for j in range(NCHUNK):
            b = j % 4
            if j >= 4:
                sc[j - 4].wait()
            ld[j] = pltpu.async_copy(
                msg_hbm.at[pl.ds(base0 + j * ch, ch)], msg_v.at[b], sl[b])
            if j >= 2:
                start_scatter(j - 2)
        for j in range(max(NCHUNK - 2, 0), NCHUNK):
            start_scatter(j)
        for j in range(max(NCHUNK - 4, 0), NCHUNK):
            sc[j].wait()
        plsc.subcore_barrier()
        pltpu.sync_copy(acc_sh.at[pl.ds(sid * NPT, NPT)],
                        out_hbm.at[cid, pl.ds(sid * NPT, NPT)])

    return pl.kernel(
        body,
        out_type=jax.ShapeDtypeStruct((NC, N, K), jnp.float32),
        mesh=_SC_MESH,
        scratch_types=[pltpu.VMEM((epw,), jnp.int32),
                       pltpu.VMEM((4, ch, K), jnp.float32),
                       pltpu.VMEM_SHARED((N, K), jnp.float32)]
        + [pltpu.SemaphoreType.DMA] * 8,
        compiler_params=_SC_PARAMS,
    )


def kernel(x, edge_index, edge_attr, batch, pos, W1, b1, W2, b2, Win, bin_,
           Wroot, broot, Wn1, bn1, Wn2, bn2, We1, be1, We2, be2):
    f32 = jnp.float32
    src = edge_index[0]
    dst = edge_index[1]
    src1, src2 = src[:EH1], src[EH1:]
    dst1, dst2 = dst[:EH1], dst[EH1:]

    xp = pl.pallas_call(
        _xp_body,
        grid=(N // TN,),
        in_specs=[pl.BlockSpec((TN, D), lambda i: (i, 0)),
                  _full((D, K)), _full((1, K))],
        out_specs=pl.BlockSpec((TN, K), lambda i: (i, 0)),
        out_shape=jax.ShapeDtypeStruct((N, K), f32),
    )(x, Win, bin_.reshape(1, K))

    pos_pad = jnp.pad(pos, ((0, 0), (0, 13)))
    tab = jnp.concatenate([xp, pos_pad], axis=1)
    gs1, gd1 = _make_gather2(2 * K, K, EH1, CH1)(tab, pos_pad, src1, dst1)
    gs2, gd2 = _make_gather2(2 * K, K, EH2, CH2)(tab, pos_pad, src2, dst2)

    r_sel = (jnp.arange(H)[None, :] // K == jnp.arange(K)[:, None]).astype(f32)
    r32_sel = jnp.pad(r_sel, ((0, K), (0, 0)))
    s_sel = (jnp.arange(H)[:, None] % K == jnp.arange(K)[None, :]).astype(f32)

    W1a = W1[:DE]
    W1b = jnp.pad(W1[DE:], ((0, 13), (0, 0)))
    W1b32 = jnp.pad(W1[DE:], ((DE, 10), (0, 0)))

    def edge_msg(ea, gs, gd, ne, te):
        return pl.pallas_call(
            _edge_msg_body,
            grid=(ne // te,),
            in_specs=[pl.BlockSpec((te, DE), lambda i: (i, 0)),
                      pl.BlockSpec((te, 2 * K), lambda i: (i, 0)),
                      pl.BlockSpec((te, K), lambda i: (i, 0)),
                      _full((DE, H)), _full((2 * K, H)), _full((K, H)),
                      _full((1, H)),
                      _full((H, K * K)), _full((1, K * K)),
                      _full((2 * K, H)), _full((H, K))],
            out_specs=pl.BlockSpec((te, K), lambda i: (i, 0)),
            out_shape=jax.ShapeDtypeStruct((ne, K), f32),
        )(ea, gs, gd, W1a, W1b32, W1b, b1.reshape(1, H), W2,
          b2.reshape(1, K * K), r32_sel, s_sel)

    msg1 = edge_msg(edge_attr[:EH1], gs1, gd1, EH1, TE1)
    msg2 = edge_msg(edge_attr[EH1:], gs2, gd2, EH2, TE2)

    zeros = jnp.zeros((N, K), f32)
    agg1 = _make_seg_sum(EH1, CH1)(msg1, dst1, zeros)
    agg2 = _make_seg_sum(EH2, CH2)(msg2, dst2, zeros)

    hn, node_pred = pl.pallas_call(
        _node_body,
        grid=(N // TN,),
        in_specs=[pl.BlockSpec((TN, K), lambda i: (i, 0)),
                  pl.BlockSpec((NC, TN, K), lambda i: (0, i, 0)),
                  pl.BlockSpec((NC, TN, K), lambda i: (0, i, 0)),
                  _full((K, K)), _full((1, K)),
                  _full((K, 64)), _full((1, 64)),
                  _full((64, 2)), _full((1, 2))],
        out_specs=[pl.BlockSpec((TN, K), lambda i: (i, 0)),
                   pl.BlockSpec((TN, 2), lambda i: (i, 0))],
        out_shape=[jax.ShapeDtypeStruct((N, K), f32),
                   jax.ShapeDtypeStruct((N, 2), f32)],
    )(xp, agg1, agg2, Wroot, broot.reshape(1, K), Wn1, bn1.reshape(1, 64),
      Wn2, bn2.reshape(1, 2))

    def edge_pred_half(hs, hd, ne, te):
        return pl.pallas_call(
            _edge_pred_body,
            grid=(ne // te,),
            in_specs=[pl.BlockSpec((te, K), lambda i: (i, 0)),
                      pl.BlockSpec((te, K), lambda i: (i, 0)),
                      _full((K, 64)), _full((K, 64)), _full((1, 64)),
                      _full((64, 2)), _full((1, 2))],
            out_specs=pl.BlockSpec((te, 2), lambda i: (i, 0)),
            out_shape=jax.ShapeDtypeStruct((ne, 2), f32),
        )(hs, hd, We1[:K], We1[K:], be1.reshape(1, 64), We2,
          be2.reshape(1, 2))

    hs1, hd1 = _make_gather2(K, K, EH1, CH1)(hn, hn, src1, dst1)
    hs2, hd2 = _make_gather2(K, K, EH2, CH2)(hn, hn, src2, dst2)
    ep1 = edge_pred_half(hs1, hd1, EH1, TE1)
    ep2 = edge_pred_half(hs2, hd2, EH2, TE2)
    edge_pred = jnp.concatenate([ep1, ep2], axis=0)

    return node_pred, edge_pred


# speedup vs baseline: 1.0332x; 1.0230x over previous
import functools

import jax
import jax.numpy as jnp
from jax import lax
from jax.experimental import pallas as pl
from jax.experimental.pallas import tpu as pltpu
from jax.experimental.pallas import tpu_sc as plsc

N, E, D, DE, H, K = 10000, 160000, 128, 16, 256, 16
EH1, EH2 = 81920, 78080
CH1, CH2 = 512, 488
TE1, TE2 = 2048, 1952
NCHUNK = 5
TN = 2000
NC, NS = 2, 16
NW = NC * NS
NPT = N // NS


def _elu(z):
    return jnp.where(z > 0, z, jnp.exp(z) - 1.0)


def _xp_body(x_ref, win_ref, bin_ref, out_ref):
    out_ref[...] = _elu(
        jnp.dot(x_ref[...], win_ref[...], preferred_element_type=jnp.float32)
        + bin_ref[...])


def _edge_msg_body(ea_ref, gs_ref, gd_ref, w1a_ref, w1b32_ref, w1b_ref,
                   b1_ref, w2_ref, b2_ref, r32_ref, s_ref, out_ref):
    gs = gs_ref[...].astype(jnp.float32)
    gd = gd_ref[...].astype(jnp.float32)
    z = (jnp.dot(ea_ref[...], w1a_ref[...], preferred_element_type=jnp.float32)
         + jnp.dot(gs, w1b32_ref[...], preferred_element_type=jnp.float32)
         - jnp.dot(gd, w1b_ref[...], preferred_element_type=jnp.float32)
         + b1_ref[...])
    h1 = _elu(z)
    kern = jnp.dot(h1, w2_ref[...], preferred_element_type=jnp.float32) + b2_ref[...]
    xrep = jnp.dot(gs, r32_ref[...], preferred_element_type=jnp.float32)
    out_ref[...] = jnp.dot(kern * xrep, s_ref[...],
                           preferred_element_type=jnp.float32)


def _node_body(xp_ref, a1_ref, a2_ref, wroot_ref, broot_ref, wn1_ref, bn1_ref,
               wn2_ref, bn2_ref, hn_ref, np_ref):
    xp = xp_ref[...]
    agg = a1_ref[0] + a1_ref[1] + a2_ref[0] + a2_ref[1]
    hn = _elu(jnp.dot(xp, wroot_ref[...], preferred_element_type=jnp.float32)
              + broot_ref[...] + agg)
    t = _elu(jnp.dot(hn, wn1_ref[...], preferred_element_type=jnp.float32)
             + bn1_ref[...])
    hn_ref[...] = hn.astype(jnp.bfloat16)
    np_ref[...] = jnp.dot(t, wn2_ref[...], preferred_element_type=jnp.float32) \
        + bn2_ref[...]


def _edge_pred_body(hs_ref, hd_ref, we1a_ref, we1b_ref, be1_ref,
                    we2_ref, be2_ref, out_ref):
    hs = hs_ref[...].astype(jnp.float32)
    hd = hd_ref[...].astype(jnp.float32)
    t = _elu(jnp.dot(hs, we1a_ref[...], preferred_element_type=jnp.float32)
             + jnp.dot(hd, we1b_ref[...], preferred_element_type=jnp.float32)
             + be1_ref[...])
    out_ref[...] = jnp.dot(t, we2_ref[...], preferred_element_type=jnp.float32) \
        + be2_ref[...]


def _full(shape):
    return pl.BlockSpec(shape, lambda i: (0,) * len(shape))


_SC_MESH = plsc.VectorSubcoreMesh(core_axis_name="c", subcore_axis_name="s")
_SC_PARAMS = pltpu.CompilerParams(use_tc_tiling_on_sc=False)


def _make_gather2(wa, wb, ne, ch):
    epw = ne // NW

    def body(taba_hbm, tabb_hbm, idxa_hbm, idxb_hbm, outa_hbm, outb_hbm,
             ia_v, ib_v, ra_v, rb_v,
             sga0, sga1, sga2, sga3, sgb0, sgb1, sgb2, sgb3,
             swa0, swa1, swa2, swa3, swb0, swb1, swb2, swb3):
        wid = lax.axis_index("s") * NC + lax.axis_index("c")
        sga = (sga0, sga1, sga2, sga3)
        sgb = (sgb0, sgb1, sgb2, sgb3)
        swa = (swa0, swa1, swa2, swa3)
        swb = (swb0, swb1, swb2, swb3)
        ga = [None] * NCHUNK
        gb = [None] * NCHUNK
        wa_ = [None] * NCHUNK
        wb_ = [None] * NCHUNK
        base0 = wid * epw
        pltpu.sync_copy(idxa_hbm.at[pl.ds(base0, epw)], ia_v)
        pltpu.sync_copy(idxb_hbm.at[pl.ds(base0, epw)], ib_v)

        def start_write(j):
            b = j % 4
            ga[j].wait()
            gb[j].wait()
            wa_[j] = pltpu.async_copy(
                ra_v.at[b], outa_hbm.at[pl.ds(base0 + j * ch, ch)], swa[b])
            wb_[j] = pltpu.async_copy(
                rb_v.at[b], outb_hbm.at[pl.ds(base0 + j * ch, ch)], swb[b])

        for j in range(NCHUNK):
            b = j % 4
            if j >= 4:
                wa_[j - 4].wait()
                wb_[j - 4].wait()
            ga[j] = pltpu.async_copy(
                taba_hbm.at[ia_v.at[pl.ds(j * ch, ch)]], ra_v.at[b], sga[b])
            gb[j] = pltpu.async_copy(
                tabb_hbm.at[ib_v.at[pl.ds(j * ch, ch)]], rb_v.at[b], sgb[b])
            if j >= 2:
                start_write(j - 2)
        for j in range(max(NCHUNK - 2, 0), NCHUNK):
            start_write(j)
        for j in range(max(NCHUNK - 4, 0), NCHUNK):
            wa_[j].wait()
            wb_[j].wait()

    return pl.kernel(
        body,
        out_type=[jax.ShapeDtypeStruct((ne, wa), jnp.bfloat16),
                  jax.ShapeDtypeStruct((ne, wb), jnp.bfloat16)],
        mesh=_SC_MESH,
        scratch_types=[pltpu.VMEM((epw,), jnp.int32),
                       pltpu.VMEM((epw,), jnp.int32),
                       pltpu.VMEM((4, ch, wa), jnp.bfloat16),
                       pltpu.VMEM((4, ch, wb), jnp.bfloat16)]
        + [pltpu.SemaphoreType.DMA] * 16,
        compiler_params=_SC_PARAMS,
    )


def _make_seg_sum(ne, ch):
    epw = ne // NW

    def body(msg_hbm, idx_hbm, zeros_hbm, out_hbm, idx_v, msg_v, acc_sh,
             sl0, sl1, sl2, sl3, ss0, ss1, ss2, ss3):
        cid = lax.axis_index("c")
        sid = lax.axis_index("s")
        wid = sid * NC + cid
        base0 = wid * epw
        pltpu.sync_copy(zeros_hbm.at[pl.ds(sid * NPT, NPT)],
                        acc_sh.at[pl.ds(sid * NPT, NPT)])
        pltpu.sync_copy(idx_hbm.at[pl.ds(base0, epw)], idx_v)
        plsc.subcore_barrier()
        sl = (sl0, sl1, sl2, sl3)
        ss = (ss0, ss1, ss2, ss3)
        ld = [None] * NCHUNK
        sc = [None] * NCHUNK

        def start_scatter(j):
            b = j % 4
            ld[j].wait()
            sc[j] = pltpu.async_copy(
                msg_v.at[b], acc_sh.at[idx_v.at[pl.ds(j * ch, ch)]], ss[b],
                add=True)

        for j in range(NCHUNK):
            b = j % 4
            if j >= 4:
                sc[j - 4].wait()
            ld[j] = pltpu.async_copy(
                msg_hbm.at[pl.ds(base0 + j * ch, ch)], msg_v.at[b], sl[b])
            if j >= 2:
                start_scatter(j - 2)
        for j in range(max(NCHUNK - 2, 0), NCHUNK):
            start_scatter(j)
        for j in range(max(NCHUNK - 4, 0), NCHUNK):
            sc[j].wait()
        plsc.subcore_barrier()
        pltpu.sync_copy(acc_sh.at[pl.ds(sid * NPT, NPT)],
                        out_hbm.at[cid, pl.ds(sid * NPT, NPT)])

    return pl.kernel(
        body,
        out_type=jax.ShapeDtypeStruct((NC, N, K), jnp.float32),
        mesh=_SC_MESH,
        scratch_types=[pltpu.VMEM((epw,), jnp.int32),
                       pltpu.VMEM((4, ch, K), jnp.float32),
                       pltpu.VMEM_SHARED((N, K), jnp.float32)]
        + [pltpu.SemaphoreType.DMA] * 8,
        compiler_params=_SC_PARAMS,
    )


def kernel(x, edge_index, edge_attr, batch, pos, W1, b1, W2, b2, Win, bin_,
           Wroot, broot, Wn1, bn1, Wn2, bn2, We1, be1, We2, be2):
    f32 = jnp.float32
    src = edge_index[0]
    dst = edge_index[1]
    src1, src2 = src[:EH1], src[EH1:]
    dst1, dst2 = dst[:EH1], dst[EH1:]

    xp = pl.pallas_call(
        _xp_body,
        grid=(N // TN,),
        in_specs=[pl.BlockSpec((TN, D), lambda i: (i, 0)),
                  _full((D, K)), _full((1, K))],
        out_specs=pl.BlockSpec((TN, K), lambda i: (i, 0)),
        out_shape=jax.ShapeDtypeStruct((N, K), f32),
    )(x, Win, bin_.reshape(1, K))

    pos_pad = jnp.pad(pos, ((0, 0), (0, 13)))
    tab = jnp.concatenate([xp, pos_pad], axis=1).astype(jnp.bfloat16)
    pos_padh = pos_pad.astype(jnp.bfloat16)
    gs1, gd1 = _make_gather2(2 * K, K, EH1, CH1)(tab, pos_padh, src1, dst1)
    gs2, gd2 = _make_gather2(2 * K, K, EH2, CH2)(tab, pos_padh, src2, dst2)

    r_sel = (jnp.arange(H)[None, :] // K == jnp.arange(K)[:, None]).astype(f32)
    r32_sel = jnp.pad(r_sel, ((0, K), (0, 0)))
    s_sel = (jnp.arange(H)[:, None] % K == jnp.arange(K)[None, :]).astype(f32)

    W1a = W1[:DE]
    W1b = jnp.pad(W1[DE:], ((0, 13), (0, 0)))
    W1b32 = jnp.pad(W1[DE:], ((DE, 10), (0, 0)))

    def edge_msg(ea, gs, gd, ne, te):
        return pl.pallas_call(
            _edge_msg_body,
            grid=(ne // te,),
            in_specs=[pl.BlockSpec((te, DE), lambda i: (i, 0)),
                      pl.BlockSpec((te, 2 * K), lambda i: (i, 0)),
                      pl.BlockSpec((te, K), lambda i: (i, 0)),
                      _full((DE, H)), _full((2 * K, H)), _full((K, H)),
                      _full((1, H)),
                      _full((H, K * K)), _full((1, K * K)),
                      _full((2 * K, H)), _full((H, K))],
            out_specs=pl.BlockSpec((te, K), lambda i: (i, 0)),
            out_shape=jax.ShapeDtypeStruct((ne, K), f32),
        )(ea, gs, gd, W1a, W1b32, W1b, b1.reshape(1, H), W2,
          b2.reshape(1, K * K), r32_sel, s_sel)

    msg1 = edge_msg(edge_attr[:EH1], gs1, gd1, EH1, TE1)
    msg2 = edge_msg(edge_attr[EH1:], gs2, gd2, EH2, TE2)

    zeros = jnp.zeros((N, K), f32)
    agg1 = _make_seg_sum(EH1, CH1)(msg1, dst1, zeros)
    agg2 = _make_seg_sum(EH2, CH2)(msg2, dst2, zeros)

    hn, node_pred = pl.pallas_call(
        _node_body,
        grid=(N // TN,),
        in_specs=[pl.BlockSpec((TN, K), lambda i: (i, 0)),
                  pl.BlockSpec((NC, TN, K), lambda i: (0, i, 0)),
                  pl.BlockSpec((NC, TN, K), lambda i: (0, i, 0)),
                  _full((K, K)), _full((1, K)),
                  _full((K, 64)), _full((1, 64)),
                  _full((64, 2)), _full((1, 2))],
        out_specs=[pl.BlockSpec((TN, K), lambda i: (i, 0)),
                   pl.BlockSpec((TN, 2), lambda i: (i, 0))],
        out_shape=[jax.ShapeDtypeStruct((N, K), jnp.bfloat16),
                   jax.ShapeDtypeStruct((N, 2), f32)],
    )(xp, agg1, agg2, Wroot, broot.reshape(1, K), Wn1, bn1.reshape(1, 64),
      Wn2, bn2.reshape(1, 2))

    def edge_pred_half(hs, hd, ne, te):
        return pl.pallas_call(
            _edge_pred_body,
            grid=(ne // te,),
            in_specs=[pl.BlockSpec((te, K), lambda i: (i, 0)),
                      pl.BlockSpec((te, K), lambda i: (i, 0)),
                      _full((K, 64)), _full((K, 64)), _full((1, 64)),
                      _full((64, 2)), _full((1, 2))],
            out_specs=pl.BlockSpec((te, 2), lambda i: (i, 0)),
            out_shape=jax.ShapeDtypeStruct((ne, 2), f32),
        )(hs, hd, We1[:K], We1[K:], be1.reshape(1, 64), We2,
          be2.reshape(1, 2))

    hs1, hd1 = _make_gather2(K, K, EH1, CH1)(hn, hn, src1, dst1)
    hs2, hd2 = _make_gather2(K, K, EH2, CH2)(hn, hn, src2, dst2)
    ep1 = edge_pred_half(hs1, hd1, EH1, TE1)
    ep2 = edge_pred_half(hs2, hd2, EH2, TE2)
    edge_pred = jnp.concatenate([ep1, ep2], axis=0)

    return node_pred, edge_pred


# speedup vs baseline: 1.1160x; 1.0802x over previous
import functools

import jax
import jax.numpy as jnp
from jax import lax
from jax.experimental import pallas as pl
from jax.experimental.pallas import tpu as pltpu
from jax.experimental.pallas import tpu_sc as plsc

N, E, D, DE, H, K = 10000, 160000, 128, 16, 256, 16
EH1, EH2 = 81920, 78080
CH1, CH2 = 512, 488
TE1, TE2 = 2048, 1952
NCHUNK = 5
TN = 2000
NC, NS = 2, 16
NW = NC * NS
NPT = N // NS


def _elu(z):
    return jnp.where(z > 0, z, jnp.exp(z) - 1.0)


def _xp_body(x_ref, win_ref, bin_ref, out_ref):
    out_ref[...] = _elu(
        jnp.dot(x_ref[...], win_ref[...], preferred_element_type=jnp.float32)
        + bin_ref[...])


def _edge_msg_body(ea_ref, gs_ref, gd_ref, w1a_ref, w1b32_ref, w1b_ref,
                   b1_ref, w2_ref, b2_ref, r32_ref, s_ref, out_ref):
    gs = gs_ref[...].astype(jnp.float32)
    gd = gd_ref[...].astype(jnp.float32)
    z = (jnp.dot(ea_ref[...], w1a_ref[...], preferred_element_type=jnp.float32)
         + jnp.dot(gs, w1b32_ref[...], preferred_element_type=jnp.float32)
         - jnp.dot(gd, w1b_ref[...], preferred_element_type=jnp.float32)
         + b1_ref[...])
    h1 = _elu(z)
    kern = jnp.dot(h1, w2_ref[...], preferred_element_type=jnp.float32) + b2_ref[...]
    xrep = jnp.dot(gs, r32_ref[...], preferred_element_type=jnp.float32)
    out_ref[...] = jnp.dot(kern * xrep, s_ref[...],
                           preferred_element_type=jnp.float32)


def _node_body(xp_ref, a1_ref, a2_ref, wroot_ref, broot_ref, wn1_ref, bn1_ref,
               wn2_ref, bn2_ref, hn_ref, np_ref):
    xp = xp_ref[...]
    agg = a1_ref[0] + a1_ref[1] + a2_ref[0] + a2_ref[1]
    hn = _elu(jnp.dot(xp, wroot_ref[...], preferred_element_type=jnp.float32)
              + broot_ref[...] + agg)
    t = _elu(jnp.dot(hn, wn1_ref[...], preferred_element_type=jnp.float32)
             + bn1_ref[...])
    hn_ref[...] = hn
    np_ref[...] = jnp.dot(t, wn2_ref[...], preferred_element_type=jnp.float32) \
        + bn2_ref[...]


def _edge_pred_body(hs_ref, hd_ref, we1a_ref, we1b_ref, be1_ref,
                    we2_ref, be2_ref, out_ref):
    t = _elu(jnp.dot(hs_ref[...], we1a_ref[...], preferred_element_type=jnp.float32)
             + jnp.dot(hd_ref[...], we1b_ref[...], preferred_element_type=jnp.float32)
             + be1_ref[...])
    out_ref[...] = jnp.dot(t, we2_ref[...], preferred_element_type=jnp.float32) \
        + be2_ref[...]


def _full(shape):
    return pl.BlockSpec(shape, lambda i: (0,) * len(shape))


_SC_MESH = plsc.VectorSubcoreMesh(core_axis_name="c", subcore_axis_name="s")
_SC_PARAMS = pltpu.CompilerParams(use_tc_tiling_on_sc=False)


def _make_gather2(wa, wb, ne, ch, dt=jnp.bfloat16):
    epw = ne // NW

    def body(taba_hbm, tabb_hbm, idxa_hbm, idxb_hbm, outa_hbm, outb_hbm,
             ia_v, ib_v, ra_v, rb_v,
             sga0, sga1, sga2, sga3, sgb0, sgb1, sgb2, sgb3,
             swa0, swa1, swa2, swa3, swb0, swb1, swb2, swb3):
        wid = lax.axis_index("s") * NC + lax.axis_index("c")
        sga = (sga0, sga1, sga2, sga3)
        sgb = (sgb0, sgb1, sgb2, sgb3)
        swa = (swa0, swa1, swa2, swa3)
        swb = (swb0, swb1, swb2, swb3)
        ga = [None] * NCHUNK
        gb = [None] * NCHUNK
        wa_ = [None] * NCHUNK
        wb_ = [None] * NCHUNK
        base0 = wid * epw
        pltpu.sync_copy(idxa_hbm.at[pl.ds(base0, epw)], ia_v)
        pltpu.sync_copy(idxb_hbm.at[pl.ds(base0, epw)], ib_v)

        def start_write(j):
            b = j % 4
            ga[j].wait()
            gb[j].wait()
            wa_[j] = pltpu.async_copy(
                ra_v.at[b], outa_hbm.at[pl.ds(base0 + j * ch, ch)], swa[b])
            wb_[j] = pltpu.async_copy(
                rb_v.at[b], outb_hbm.at[pl.ds(base0 + j * ch, ch)], swb[b])

        for j in range(NCHUNK):
            b = j % 4
            if j >= 4:
                wa_[j - 4].wait()
                wb_[j - 4].wait()
            ga[j] = pltpu.async_copy(
                taba_hbm.at[ia_v.at[pl.ds(j * ch, ch)]], ra_v.at[b], sga[b])
            gb[j] = pltpu.async_copy(
                tabb_hbm.at[ib_v.at[pl.ds(j * ch, ch)]], rb_v.at[b], sgb[b])
            if j >= 2:
                start_write(j - 2)
        for j in range(max(NCHUNK - 2, 0), NCHUNK):
            start_write(j)
        for j in range(max(NCHUNK - 4, 0), NCHUNK):
            wa_[j].wait()
            wb_[j].wait()

    return pl.kernel(
        body,
        out_type=[jax.ShapeDtypeStruct((ne, wa), dt),
                  jax.ShapeDtypeStruct((ne, wb), dt)],
        mesh=_SC_MESH,
        scratch_types=[pltpu.VMEM((epw,), jnp.int32),
                       pltpu.VMEM((epw,), jnp.int32),
                       pltpu.VMEM((4, ch, wa), dt),
                       pltpu.VMEM((4, ch, wb), dt)]
        + [pltpu.SemaphoreType.DMA] * 16,
        compiler_params=_SC_PARAMS,
    )


def _make_seg_sum(ne, ch):
    epw = ne // NW

    def body(msg_hbm, idx_hbm, zeros_hbm, out_hbm, idx_v, msg_v, acc_sh,
             sl0, sl1, sl2, sl3, ss0, ss1, ss2, ss3):
        cid = lax.axis_index("c")
        sid = lax.axis_index("s")
        wid = sid * NC + cid
        base0 = wid * epw
        pltpu.sync_copy(zeros_hbm.at[pl.ds(sid * NPT, NPT)],
                        acc_sh.at[pl.ds(sid * NPT, NPT)])
        pltpu.sync_copy(idx_hbm.at[pl.ds(base0, epw)], idx_v)
        plsc.subcore_barrier()
        sl = (sl0, sl1, sl2, sl3)
        ss = (ss0, ss1, ss2, ss3)
        ld = [None] * NCHUNK
        sc = [None] * NCHUNK

        def start_scatter(j):
            b = j % 4
            ld[j].wait()
            sc[j] = pltpu.async_copy(
                msg_v.at[b], acc_sh.at[idx_v.at[pl.ds(j * ch, ch)]], ss[b],
                add=True)

        for j in range(NCHUNK):
            b = j % 4
            if j >= 4:
                sc[j - 4].wait()
            ld[j] = pltpu.async_copy(
                msg_hbm.at[pl.ds(base0 + j * ch, ch)], msg_v.at[b], sl[b])
            if j >= 2:
                start_scatter(j - 2)
        for j in range(max(NCHUNK - 2, 0), NCHUNK):
            start_scatter(j)
        for j in range(max(NCHUNK - 4, 0), NCHUNK):
            sc[j].wait()
        plsc.subcore_barrier()
        pltpu.sync_copy(acc_sh.at[pl.ds(sid * NPT, NPT)],
                        out_hbm.at[cid, pl.ds(sid * NPT, NPT)])

    return pl.kernel(
        body,
        out_type=jax.ShapeDtypeStruct((NC, N, K), jnp.float32),
        mesh=_SC_MESH,
        scratch_types=[pltpu.VMEM((epw,), jnp.int32),
                       pltpu.VMEM((4, ch, K), jnp.float32),
                       pltpu.VMEM_SHARED((N, K), jnp.float32)]
        + [pltpu.SemaphoreType.DMA] * 8,
        compiler_params=_SC_PARAMS,
    )


def kernel(x, edge_index, edge_attr, batch, pos, W1, b1, W2, b2, Win, bin_,
           Wroot, broot, Wn1, bn1, Wn2, bn2, We1, be1, We2, be2):
    f32 = jnp.float32
    src = edge_index[0]
    dst = edge_index[1]
    src1, src2 = src[:EH1], src[EH1:]
    dst1, dst2 = dst[:EH1], dst[EH1:]

    xp = pl.pallas_call(
        _xp_body,
        grid=(N // TN,),
        in_specs=[pl.BlockSpec((TN, D), lambda i: (i, 0)),
                  _full((D, K)), _full((1, K))],
        out_specs=pl.BlockSpec((TN, K), lambda i: (i, 0)),
        out_shape=jax.ShapeDtypeStruct((N, K), f32),
    )(x, Win, bin_.reshape(1, K))

    pos_pad = jnp.pad(pos, ((0, 0), (0, 13)))
    tab = jnp.concatenate([xp, pos_pad], axis=1).astype(jnp.bfloat16)
    pos_padh = pos_pad.astype(jnp.bfloat16)
    gs1, gd1 = _make_gather2(2 * K, K, EH1, CH1)(tab, pos_padh, src1, dst1)
    gs2, gd2 = _make_gather2(2 * K, K, EH2, CH2)(tab, pos_padh, src2, dst2)

    r_sel = (jnp.arange(H)[None, :] // K == jnp.arange(K)[:, None]).astype(f32)
    r32_sel = jnp.pad(r_sel, ((0, K), (0, 0)))
    s_sel = (jnp.arange(H)[:, None] % K == jnp.arange(K)[None, :]).astype(f32)

    W1a = W1[:DE]
    W1b = jnp.pad(W1[DE:], ((0, 13), (0, 0)))
    W1b32 = jnp.pad(W1[DE:], ((DE, 10), (0, 0)))

    def edge_msg(ea, gs, gd, ne, te):
        return pl.pallas_call(
            _edge_msg_body,
            grid=(ne // te,),
            in_specs=[pl.BlockSpec((te, DE), lambda i: (i, 0)),
                      pl.BlockSpec((te, 2 * K), lambda i: (i, 0)),
                      pl.BlockSpec((te, K), lambda i: (i, 0)),
                      _full((DE, H)), _full((2 * K, H)), _full((K, H)),
                      _full((1, H)),
                      _full((H, K * K)), _full((1, K * K)),
                      _full((2 * K, H)), _full((H, K))],
            out_specs=pl.BlockSpec((te, K), lambda i: (i, 0)),
            out_shape=jax.ShapeDtypeStruct((ne, K), f32),
        )(ea, gs, gd, W1a, W1b32, W1b, b1.reshape(1, H), W2,
          b2.reshape(1, K * K), r32_sel, s_sel)

    msg1 = edge_msg(edge_attr[:EH1], gs1, gd1, EH1, TE1)
    msg2 = edge_msg(edge_attr[EH1:], gs2, gd2, EH2, TE2)

    zeros = jnp.zeros((N, K), f32)
    agg1 = _make_seg_sum(EH1, CH1)(msg1, dst1, zeros)
    agg2 = _make_seg_sum(EH2, CH2)(msg2, dst2, zeros)

    hn, node_pred = pl.pallas_call(
        _node_body,
        grid=(N // TN,),
        in_specs=[pl.BlockSpec((TN, K), lambda i: (i, 0)),
                  pl.BlockSpec((NC, TN, K), lambda i: (0, i, 0)),
                  pl.BlockSpec((NC, TN, K), lambda i: (0, i, 0)),
                  _full((K, K)), _full((1, K)),
                  _full((K, 64)), _full((1, 64)),
                  _full((64, 2)), _full((1, 2))],
        out_specs=[pl.BlockSpec((TN, K), lambda i: (i, 0)),
                   pl.BlockSpec((TN, 2), lambda i: (i, 0))],
        out_shape=[jax.ShapeDtypeStruct((N, K), f32),
                   jax.ShapeDtypeStruct((N, 2), f32)],
    )(xp, agg1, agg2, Wroot, broot.reshape(1, K), Wn1, bn1.reshape(1, 64),
      Wn2, bn2.reshape(1, 2))

    bd_a = jnp.kron(jnp.eye(8, dtype=f32), We1[:K])
    bd_b = jnp.kron(jnp.eye(8, dtype=f32), We1[K:])
    bd_w2 = jnp.kron(jnp.eye(8, dtype=f32), We2)
    be1_8 = jnp.tile(be1, 8).reshape(1, 512)
    be2_8 = jnp.tile(be2, 8).reshape(1, 16)

    def edge_pred_half(hs, hd, ne, te):
        tp = te // 8
        return pl.pallas_call(
            _edge_pred_body,
            grid=(ne // te,),
            in_specs=[pl.BlockSpec((tp, 128), lambda i: (i, 0)),
                      pl.BlockSpec((tp, 128), lambda i: (i, 0)),
                      _full((128, 512)), _full((128, 512)), _full((1, 512)),
                      _full((512, 16)), _full((1, 16))],
            out_specs=pl.BlockSpec((tp, 16), lambda i: (i, 0)),
            out_shape=jax.ShapeDtypeStruct((ne // 8, 16), f32),
        )(hs, hd, bd_a, bd_b, be1_8, bd_w2, be2_8)

    hs1, hd1 = _make_gather2(K, K, EH1, CH1, jnp.float32)(hn, hn, src1, dst1)
    hs2, hd2 = _make_gather2(K, K, EH2, CH2, jnp.float32)(hn, hn, src2, dst2)
    ep1 = edge_pred_half(hs1.reshape(EH1 // 8, 128),
                         hd1.reshape(EH1 // 8, 128), EH1, TE1)
    ep2 = edge_pred_half(hs2.reshape(EH2 // 8, 128),
                         hd2.reshape(EH2 // 8, 128), EH2, 1280)
    edge_pred = jnp.concatenate([ep1.reshape(EH1, 2), ep2.reshape(EH2, 2)],
                                axis=0)

    return node_pred, edge_pred
